# Initial kernel scaffold; baseline (speedup 1.0000x reference)
import jax
import jax.numpy as jnp
from jax.experimental import pallas as pl


def kernel(g1_w, g1_as, g1_ad, g1_b, g2_w, g2_as, g2_ad, g2_b, g3_w, g3_as, g3_ad, g3_b, te_w1, te_b1, te_w2, te_b2, fc1_w, fc1_b, fc2_w, fc2_b, res_w, res_b, ln_g, ln_b, x, edge_index, edge_time):
    raise NotImplementedError("write your pallas kernel here")


# speedup vs baseline: 1.2136x; 1.2136x over previous
import functools

import jax
import jax.numpy as jnp
from jax import lax
from jax.experimental import pallas as pl
from jax.experimental.pallas import tpu as pltpu


_VMEM_LIMIT = 64 * 1024 * 1024
_ROW_TILE = 256
_PROJ_ROW_TILE = 512


def _proj_kernel(x_ref, w_ref, o_ref):
    o_ref[0] = jnp.dot(x_ref[...].astype(jnp.bfloat16), w_ref[...],
                       preferred_element_type=jnp.float32).astype(o_ref.dtype)


def _project(x, w, *, heads, head_dim):
    M, K = x.shape
    tm = M if M <= _PROJ_ROW_TILE else _PROJ_ROW_TILE
    return pl.pallas_call(
        _proj_kernel,
        out_shape=jax.ShapeDtypeStruct((heads, M, head_dim), jnp.bfloat16),
        grid=(pl.cdiv(M, tm), heads),
        in_specs=[
            pl.BlockSpec((tm, K), lambda i, hh: (i, 0)),
            pl.BlockSpec((K, head_dim), lambda i, hh: (0, hh)),
        ],
        out_specs=pl.BlockSpec((1, tm, head_dim), lambda i, hh: (hh, i, 0)),
        compiler_params=pltpu.CompilerParams(
            dimension_semantics=("parallel", "arbitrary"),
            vmem_limit_bytes=_VMEM_LIMIT),
    )(x, w.astype(jnp.bfloat16))


def _gat_head_body(hs, hd, asrc, adst, adj):
    a_dst = jnp.sum(hd.astype(jnp.float32) * adst, axis=-1, keepdims=True)
    a_src = lax.dot_general(asrc.astype(jnp.bfloat16), hs,
                            (((1,), (1,)), ((), ())),
                            preferred_element_type=jnp.float32)
    mx = jnp.max(a_src)
    admx = a_dst + mx
    c = jnp.where(admx > 0, admx, 0.2 * admx)
    rp = jnp.exp(admx - c)
    rn = jnp.exp(0.2 * admx - c)
    qp = jnp.exp(a_src - mx)
    qn = jnp.exp(0.2 * (a_src - mx))
    z = a_dst + a_src
    w = jnp.where(z > 0, rp * qp, rn * qn) * adj.astype(jnp.float32)
    rowsum = jnp.sum(w, axis=1, keepdims=True)
    acc = jnp.dot(w.astype(jnp.bfloat16), hs,
                  preferred_element_type=jnp.float32)
    return acc * pl.reciprocal(rowsum, approx=False)


def _gat_concat_kernel(hsrc_ref, hdst_ref, asrc_ref, adst_ref, adj_ref,
                       bias_ref, o_ref):
    hh = pl.program_id(1)
    out = _gat_head_body(hsrc_ref[hh], hdst_ref[0], asrc_ref[0], adst_ref[0],
                         adj_ref[...])
    o_ref[...] = jnp.maximum(out + bias_ref[0], 0.0).astype(o_ref.dtype)


def _gat_mean_kernel(hsrc_ref, hdst_ref, asrc_ref, adst_ref, adj_ref,
                     bias_ref, o_ref, acc_ref, *, heads):
    hh = pl.program_id(1)

    @pl.when(hh == 0)
    def _():
        acc_ref[...] = jnp.zeros_like(acc_ref)

    acc_ref[...] += _gat_head_body(hsrc_ref[hh], hdst_ref[0], asrc_ref[0],
                                   adst_ref[0], adj_ref[...])

    @pl.when(hh == heads - 1)
    def _():
        o_ref[...] = jnp.maximum(acc_ref[...] * (1.0 / heads) + bias_ref[...],
                                 0.0)


def _gat_concat(h, att_src, att_dst, bias, adj):
    heads, N, C = h.shape
    TM = N if N <= _ROW_TILE else _ROW_TILE
    R = pl.cdiv(N, TM)
    return pl.pallas_call(
        _gat_concat_kernel,
        out_shape=jax.ShapeDtypeStruct((N, heads * C), jnp.bfloat16),
        grid=(R, heads),
        in_specs=[
            pl.BlockSpec((heads, N, C), lambda i, hh: (0, 0, 0)),
            pl.BlockSpec((1, TM, C), lambda i, hh: (hh, i, 0)),
            pl.BlockSpec((1, 1, C), lambda i, hh: (hh, 0, 0)),
            pl.BlockSpec((1, 1, C), lambda i, hh: (hh, 0, 0)),
            pl.BlockSpec((TM, N), lambda i, hh: (i, 0)),
            pl.BlockSpec((1, 1, C), lambda i, hh: (hh, 0, 0)),
        ],
        out_specs=pl.BlockSpec((TM, C), lambda i, hh: (i, hh)),
        compiler_params=pltpu.CompilerParams(
            dimension_semantics=("parallel", "arbitrary"),
            vmem_limit_bytes=_VMEM_LIMIT),
    )(h, h, att_src.reshape(heads, 1, C), att_dst.reshape(heads, 1, C), adj,
      bias.reshape(heads, 1, C))


def _gat_mean(h, att_src, att_dst, bias, adj):
    heads, N, C = h.shape
    TM = N if N <= _ROW_TILE else _ROW_TILE
    R = pl.cdiv(N, TM)
    return pl.pallas_call(
        functools.partial(_gat_mean_kernel, heads=heads),
        out_shape=jax.ShapeDtypeStruct((N, C), jnp.float32),
        grid=(R, heads),
        in_specs=[
            pl.BlockSpec((heads, N, C), lambda i, hh: (0, 0, 0)),
            pl.BlockSpec((1, TM, C), lambda i, hh: (hh, i, 0)),
            pl.BlockSpec((1, 1, C), lambda i, hh: (hh, 0, 0)),
            pl.BlockSpec((1, 1, C), lambda i, hh: (hh, 0, 0)),
            pl.BlockSpec((TM, N), lambda i, hh: (i, 0)),
            pl.BlockSpec((1, C), lambda i, hh: (0, 0)),
        ],
        out_specs=pl.BlockSpec((TM, C), lambda i, hh: (i, 0)),
        scratch_shapes=[pltpu.VMEM((TM, C), jnp.float32)],
        compiler_params=pltpu.CompilerParams(
            dimension_semantics=("parallel", "arbitrary"),
            vmem_limit_bytes=_VMEM_LIMIT),
    )(h, h, att_src.reshape(heads, 1, C), att_dst.reshape(heads, 1, C), adj,
      bias.reshape(1, C))


def _temporal_kernel(t_ref, w1_ref, b1_ref, w2_ref, b2_ref, o_ref, acc_ref,
                     *, n_edges, tile_e):
    i = pl.program_id(0)

    @pl.when(i == 0)
    def _():
        acc_ref[...] = jnp.zeros_like(acc_ref)

    t = t_ref[...]
    hdn = jnp.maximum(t * w1_ref[...] + b1_ref[...], 0.0)
    rid = i * tile_e + lax.broadcasted_iota(jnp.int32, hdn.shape, 0)
    hdn = jnp.where(rid < n_edges, hdn, 0.0)
    acc_ref[...] += jnp.sum(hdn, axis=0, keepdims=True)

    @pl.when(i == pl.num_programs(0) - 1)
    def _():
        o_ref[...] = jnp.dot(acc_ref[...] * (1.0 / n_edges), w2_ref[...],
                             preferred_element_type=jnp.float32) + b2_ref[...]


def _temporal_mean(t, w1, b1, w2, b2):
    E = t.shape[0]
    Hd = w1.shape[1]
    O = w2.shape[1]
    TE = E if E <= 8192 else 8192
    return pl.pallas_call(
        functools.partial(_temporal_kernel, n_edges=E, tile_e=TE),
        out_shape=jax.ShapeDtypeStruct((1, O), jnp.float32),
        grid=(pl.cdiv(E, TE),),
        in_specs=[
            pl.BlockSpec((TE, 1), lambda i: (i, 0)),
            pl.BlockSpec((1, Hd), lambda i: (0, 0)),
            pl.BlockSpec((1, Hd), lambda i: (0, 0)),
            pl.BlockSpec((Hd, O), lambda i: (0, 0)),
            pl.BlockSpec((1, O), lambda i: (0, 0)),
        ],
        out_specs=pl.BlockSpec((1, O), lambda i: (0, 0)),
        scratch_shapes=[pltpu.VMEM((1, Hd), jnp.float32)],
        compiler_params=pltpu.CompilerParams(
            dimension_semantics=("arbitrary",), vmem_limit_bytes=_VMEM_LIMIT),
    )(t, w1, b1.reshape(1, Hd), w2, b2.reshape(1, O))


def _tail_kernel(adjt_ref, h3_ref, h3row_ref, x_ref, time_ref, resw_ref,
                 resb_ref, w1a_ref, w1b_ref, b1_ref, w2_ref, b2_ref, g_ref,
                 beta_ref, o_ref):
    h3b = h3_ref[...].astype(jnp.bfloat16)
    x_agg = jnp.dot(adjt_ref[...], h3b,
                    preferred_element_type=jnp.float32) - h3row_ref[...]

    x_res = jnp.dot(x_ref[...], resw_ref[...],
                    preferred_element_type=jnp.float32) + resb_ref[...]

    t_contrib = jnp.dot(time_ref[...], w1b_ref[...],
                        preferred_element_type=jnp.float32)
    z = jnp.dot(x_agg, w1a_ref[...], preferred_element_type=jnp.float32)
    z = jnp.maximum(z + t_contrib + b1_ref[...], 0.0)
    z = jnp.dot(z, w2_ref[...], preferred_element_type=jnp.float32) + b2_ref[...]

    mu = jnp.mean(z, axis=-1, keepdims=True)
    var = jnp.mean((z - mu) ** 2, axis=-1, keepdims=True)
    z = (z - mu) * lax.rsqrt(var + 1e-5) * g_ref[...] + beta_ref[...]
    o_ref[...] = z + x_res


def _fused_tail(adjt, h3, x, time_mean, p):
    N, in_c = x.shape
    out_c = p['fc2_w'].shape[1]
    TM = N if N <= _ROW_TILE else _ROW_TILE
    R = pl.cdiv(N, TM)
    w1a = p['fc1_w'][:out_c]
    w1b = p['fc1_w'][out_c:]
    r2 = lambda a: a.reshape(1, -1)

    def full2(shape):
        return pl.BlockSpec(shape, lambda i: (0, 0))

    return pl.pallas_call(
        _tail_kernel,
        out_shape=jax.ShapeDtypeStruct((N, out_c), jnp.float32),
        grid=(R,),
        in_specs=[
            pl.BlockSpec((TM, N), lambda i: (i, 0)),
            full2((N, out_c)),
            pl.BlockSpec((TM, out_c), lambda i: (i, 0)),
            pl.BlockSpec((TM, in_c), lambda i: (i, 0)),
            full2((1, out_c)),
            full2((in_c, out_c)),
            full2((1, out_c)),
            full2((out_c, out_c)),
            full2((out_c, out_c)),
            full2((1, out_c)),
            full2((out_c, out_c)),
            full2((1, out_c)),
            full2((1, out_c)),
            full2((1, out_c)),
        ],
        out_specs=pl.BlockSpec((TM, out_c), lambda i: (i, 0)),
        compiler_params=pltpu.CompilerParams(
            dimension_semantics=("parallel",), vmem_limit_bytes=_VMEM_LIMIT),
    )(adjt, h3, h3, x, time_mean, p['res_w'], r2(p['res_b']),
      w1a, w1b, r2(p['fc1_b']), p['fc2_w'], r2(p['fc2_b']),
      r2(p['ln_g']), r2(p['ln_b']))


def kernel(g1_w, g1_as, g1_ad, g1_b, g2_w, g2_as, g2_ad, g2_b,
           g3_w, g3_as, g3_ad, g3_b, te_w1, te_b1, te_w2, te_b2,
           fc1_w, fc1_b, fc2_w, fc2_b, res_w, res_b, ln_g, ln_b,
           x, edge_index, edge_time):
    N = x.shape[0]
    heads, hidden = 4, 128
    out_c = fc2_w.shape[1]
    p = {'fc1_w': fc1_w, 'fc1_b': fc1_b, 'fc2_w': fc2_w, 'fc2_b': fc2_b,
         'res_w': res_w, 'res_b': res_b, 'ln_g': ln_g, 'ln_b': ln_b}

    src, dst = edge_index[0], edge_index[1]
    diag = jnp.arange(N)
    adj = (jnp.zeros((N, N), jnp.float32).at[dst, src].add(1.0)
           .at[diag, diag].add(2.0)).astype(jnp.bfloat16)
    adjt = adj.T

    t = edge_time.astype(jnp.float32).reshape(-1, 1)
    time_mean = _temporal_mean(t, te_w1, te_b1, te_w2, te_b2)

    h = _project(x, g1_w, heads=heads, head_dim=hidden)
    y = _gat_concat(h, g1_as, g1_ad, g1_b, adj)
    h = _project(y, g2_w, heads=heads, head_dim=hidden)
    y = _gat_concat(h, g2_as, g2_ad, g2_b, adj)
    h = _project(y, g3_w, heads=heads, head_dim=out_c)
    h3 = _gat_mean(h, g3_as, g3_ad, g3_b, adj)

    return _fused_tail(adjt, h3, x, time_mean, p)


# speedup vs baseline: 1.2917x; 1.0643x over previous
import functools

import jax
import jax.numpy as jnp
from jax import lax
from jax.experimental import pallas as pl
from jax.experimental.pallas import tpu as pltpu


_VMEM_LIMIT = 64 * 1024 * 1024
_ROW_TILE = 256
_PROJ_ROW_TILE = 512


def _proj_kernel(x_ref, w_ref, o_ref):
    o_ref[0] = jnp.dot(x_ref[...].astype(jnp.bfloat16), w_ref[...],
                       preferred_element_type=jnp.float32).astype(o_ref.dtype)


def _project(x, w, *, heads, head_dim):
    M, K = x.shape
    tm = M if M <= _PROJ_ROW_TILE else _PROJ_ROW_TILE
    return pl.pallas_call(
        _proj_kernel,
        out_shape=jax.ShapeDtypeStruct((heads, M, head_dim), jnp.bfloat16),
        grid=(pl.cdiv(M, tm), heads),
        in_specs=[
            pl.BlockSpec((tm, K), lambda i, hh: (i, 0)),
            pl.BlockSpec((K, head_dim), lambda i, hh: (0, hh)),
        ],
        out_specs=pl.BlockSpec((1, tm, head_dim), lambda i, hh: (hh, i, 0)),
        compiler_params=pltpu.CompilerParams(
            dimension_semantics=("parallel", "arbitrary"),
            vmem_limit_bytes=_VMEM_LIMIT),
    )(x, w.astype(jnp.bfloat16))


def _gat_head_body(hs, hd, asrc, adst, adj):
    a_dst = jnp.sum(hd.astype(jnp.float32) * adst, axis=-1, keepdims=True)
    a_src = lax.dot_general(asrc.astype(jnp.bfloat16), hs,
                            (((1,), (1,)), ((), ())),
                            preferred_element_type=jnp.float32)
    mx = jnp.max(a_src)
    admx = a_dst + mx
    c = jnp.where(admx > 0, admx, 0.2 * admx)
    rp = jnp.exp(admx - c)
    rn = jnp.exp(0.2 * admx - c)
    qp = jnp.exp(a_src - mx)
    qn = jnp.exp(0.2 * (a_src - mx))
    w = jnp.maximum(rp * qp, rn * qn) * adj.astype(jnp.float32)
    rowsum = jnp.sum(w, axis=1, keepdims=True)
    acc = jnp.dot(w.astype(jnp.bfloat16), hs,
                  preferred_element_type=jnp.float32)
    return acc * pl.reciprocal(rowsum, approx=False)


def _gat_concat_kernel(hsrc_ref, hdst_ref, asrc_ref, adst_ref, adj_ref,
                       bias_ref, o_ref):
    hh = pl.program_id(1)
    out = _gat_head_body(hsrc_ref[hh], hdst_ref[0], asrc_ref[0], adst_ref[0],
                         adj_ref[...])
    o_ref[...] = jnp.maximum(out + bias_ref[0], 0.0).astype(o_ref.dtype)


def _gat_mean_kernel(hsrc_ref, hdst_ref, asrc_ref, adst_ref, adj_ref,
                     bias_ref, o_ref, acc_ref, *, heads):
    hh = pl.program_id(1)

    @pl.when(hh == 0)
    def _():
        acc_ref[...] = jnp.zeros_like(acc_ref)

    acc_ref[...] += _gat_head_body(hsrc_ref[hh], hdst_ref[0], asrc_ref[0],
                                   adst_ref[0], adj_ref[...])

    @pl.when(hh == heads - 1)
    def _():
        o_ref[...] = jnp.maximum(acc_ref[...] * (1.0 / heads) + bias_ref[...],
                                 0.0)


def _gat_concat(h, att_src, att_dst, bias, adj):
    heads, N, C = h.shape
    TM = N if N <= _ROW_TILE else _ROW_TILE
    R = pl.cdiv(N, TM)
    return pl.pallas_call(
        _gat_concat_kernel,
        out_shape=jax.ShapeDtypeStruct((N, heads * C), jnp.bfloat16),
        grid=(R, heads),
        in_specs=[
            pl.BlockSpec((heads, N, C), lambda i, hh: (0, 0, 0)),
            pl.BlockSpec((1, TM, C), lambda i, hh: (hh, i, 0)),
            pl.BlockSpec((1, 1, C), lambda i, hh: (hh, 0, 0)),
            pl.BlockSpec((1, 1, C), lambda i, hh: (hh, 0, 0)),
            pl.BlockSpec((TM, N), lambda i, hh: (i, 0)),
            pl.BlockSpec((1, 1, C), lambda i, hh: (hh, 0, 0)),
        ],
        out_specs=pl.BlockSpec((TM, C), lambda i, hh: (i, hh)),
        compiler_params=pltpu.CompilerParams(
            dimension_semantics=("parallel", "arbitrary"),
            vmem_limit_bytes=_VMEM_LIMIT),
    )(h, h, att_src.reshape(heads, 1, C), att_dst.reshape(heads, 1, C), adj,
      bias.reshape(heads, 1, C))


def _gat_mean(h, att_src, att_dst, bias, adj):
    heads, N, C = h.shape
    TM = N if N <= _ROW_TILE else _ROW_TILE
    R = pl.cdiv(N, TM)
    return pl.pallas_call(
        functools.partial(_gat_mean_kernel, heads=heads),
        out_shape=jax.ShapeDtypeStruct((N, C), jnp.float32),
        grid=(R, heads),
        in_specs=[
            pl.BlockSpec((heads, N, C), lambda i, hh: (0, 0, 0)),
            pl.BlockSpec((1, TM, C), lambda i, hh: (hh, i, 0)),
            pl.BlockSpec((1, 1, C), lambda i, hh: (hh, 0, 0)),
            pl.BlockSpec((1, 1, C), lambda i, hh: (hh, 0, 0)),
            pl.BlockSpec((TM, N), lambda i, hh: (i, 0)),
            pl.BlockSpec((1, C), lambda i, hh: (0, 0)),
        ],
        out_specs=pl.BlockSpec((TM, C), lambda i, hh: (i, 0)),
        scratch_shapes=[pltpu.VMEM((TM, C), jnp.float32)],
        compiler_params=pltpu.CompilerParams(
            dimension_semantics=("parallel", "arbitrary"),
            vmem_limit_bytes=_VMEM_LIMIT),
    )(h, h, att_src.reshape(heads, 1, C), att_dst.reshape(heads, 1, C), adj,
      bias.reshape(1, C))


def _temporal_kernel(t_ref, w1_ref, b1_ref, w2_ref, b2_ref, o_ref, acc_ref,
                     *, n_edges, tile_e):
    i = pl.program_id(0)

    @pl.when(i == 0)
    def _():
        acc_ref[...] = jnp.zeros_like(acc_ref)

    t = t_ref[...]
    hdn = jnp.maximum(t * w1_ref[...] + b1_ref[...], 0.0)
    rid = i * tile_e + lax.broadcasted_iota(jnp.int32, hdn.shape, 0)
    hdn = jnp.where(rid < n_edges, hdn, 0.0)
    acc_ref[...] += jnp.sum(hdn, axis=0, keepdims=True)

    @pl.when(i == pl.num_programs(0) - 1)
    def _():
        o_ref[...] = jnp.dot(acc_ref[...] * (1.0 / n_edges), w2_ref[...],
                             preferred_element_type=jnp.float32) + b2_ref[...]


def _temporal_mean(t, w1, b1, w2, b2):
    E = t.shape[0]
    Hd = w1.shape[1]
    O = w2.shape[1]
    TE = E if E <= 8192 else 8192
    return pl.pallas_call(
        functools.partial(_temporal_kernel, n_edges=E, tile_e=TE),
        out_shape=jax.ShapeDtypeStruct((1, O), jnp.float32),
        grid=(pl.cdiv(E, TE),),
        in_specs=[
            pl.BlockSpec((TE, 1), lambda i: (i, 0)),
            pl.BlockSpec((1, Hd), lambda i: (0, 0)),
            pl.BlockSpec((1, Hd), lambda i: (0, 0)),
            pl.BlockSpec((Hd, O), lambda i: (0, 0)),
            pl.BlockSpec((1, O), lambda i: (0, 0)),
        ],
        out_specs=pl.BlockSpec((1, O), lambda i: (0, 0)),
        scratch_shapes=[pltpu.VMEM((1, Hd), jnp.float32)],
        compiler_params=pltpu.CompilerParams(
            dimension_semantics=("arbitrary",), vmem_limit_bytes=_VMEM_LIMIT),
    )(t, w1, b1.reshape(1, Hd), w2, b2.reshape(1, O))


def _tail_kernel(adjt_ref, h3_ref, h3row_ref, x_ref, time_ref, resw_ref,
                 resb_ref, w1a_ref, w1b_ref, b1_ref, w2_ref, b2_ref, g_ref,
                 beta_ref, o_ref):
    h3b = h3_ref[...].astype(jnp.bfloat16)
    x_agg = jnp.dot(adjt_ref[...], h3b,
                    preferred_element_type=jnp.float32) - h3row_ref[...]

    x_res = jnp.dot(x_ref[...].astype(jnp.bfloat16), resw_ref[...],
                    preferred_element_type=jnp.float32) + resb_ref[...]

    t_contrib = jnp.dot(time_ref[...], w1b_ref[...],
                        preferred_element_type=jnp.float32)
    z = jnp.dot(x_agg, w1a_ref[...], preferred_element_type=jnp.float32)
    z = jnp.maximum(z + t_contrib + b1_ref[...], 0.0)
    z = jnp.dot(z, w2_ref[...], preferred_element_type=jnp.float32) + b2_ref[...]

    mu = jnp.mean(z, axis=-1, keepdims=True)
    var = jnp.mean((z - mu) ** 2, axis=-1, keepdims=True)
    z = (z - mu) * lax.rsqrt(var + 1e-5) * g_ref[...] + beta_ref[...]
    o_ref[...] = z + x_res


def _fused_tail(adjt, h3, x, time_mean, p):
    N, in_c = x.shape
    out_c = p['fc2_w'].shape[1]
    TM = N if N <= _ROW_TILE else _ROW_TILE
    R = pl.cdiv(N, TM)
    w1a = p['fc1_w'][:out_c]
    w1b = p['fc1_w'][out_c:]
    r2 = lambda a: a.reshape(1, -1)

    def full2(shape):
        return pl.BlockSpec(shape, lambda i: (0, 0))

    return pl.pallas_call(
        _tail_kernel,
        out_shape=jax.ShapeDtypeStruct((N, out_c), jnp.float32),
        grid=(R,),
        in_specs=[
            pl.BlockSpec((TM, N), lambda i: (i, 0)),
            full2((N, out_c)),
            pl.BlockSpec((TM, out_c), lambda i: (i, 0)),
            pl.BlockSpec((TM, in_c), lambda i: (i, 0)),
            full2((1, out_c)),
            full2((in_c, out_c)),
            full2((1, out_c)),
            full2((out_c, out_c)),
            full2((out_c, out_c)),
            full2((1, out_c)),
            full2((out_c, out_c)),
            full2((1, out_c)),
            full2((1, out_c)),
            full2((1, out_c)),
        ],
        out_specs=pl.BlockSpec((TM, out_c), lambda i: (i, 0)),
        compiler_params=pltpu.CompilerParams(
            dimension_semantics=("parallel",), vmem_limit_bytes=_VMEM_LIMIT),
    )(adjt, h3, h3, x, time_mean, p['res_w'].astype(jnp.bfloat16), r2(p['res_b']),
      w1a, w1b, r2(p['fc1_b']), p['fc2_w'], r2(p['fc2_b']),
      r2(p['ln_g']), r2(p['ln_b']))


def kernel(g1_w, g1_as, g1_ad, g1_b, g2_w, g2_as, g2_ad, g2_b,
           g3_w, g3_as, g3_ad, g3_b, te_w1, te_b1, te_w2, te_b2,
           fc1_w, fc1_b, fc2_w, fc2_b, res_w, res_b, ln_g, ln_b,
           x, edge_index, edge_time):
    N = x.shape[0]
    heads, hidden = 4, 128
    out_c = fc2_w.shape[1]
    p = {'fc1_w': fc1_w, 'fc1_b': fc1_b, 'fc2_w': fc2_w, 'fc2_b': fc2_b,
         'res_w': res_w, 'res_b': res_b, 'ln_g': ln_g, 'ln_b': ln_b}

    src, dst = edge_index[0], edge_index[1]
    diag = jnp.arange(N, dtype=edge_index.dtype)
    rows = jnp.concatenate([dst, diag])
    cols = jnp.concatenate([src, diag])
    upd = jnp.concatenate([jnp.ones(src.shape[0], jnp.float32),
                           jnp.full((N,), 2.0, jnp.float32)])
    adj = jnp.zeros((N, N), jnp.float32).at[rows, cols].add(upd).astype(jnp.bfloat16)
    adjt = adj.T

    t = edge_time.astype(jnp.float32).reshape(-1, 1)
    time_mean = _temporal_mean(t, te_w1, te_b1, te_w2, te_b2)

    h = _project(x, g1_w, heads=heads, head_dim=hidden)
    y = _gat_concat(h, g1_as, g1_ad, g1_b, adj)
    h = _project(y, g2_w, heads=heads, head_dim=hidden)
    y = _gat_concat(h, g2_as, g2_ad, g2_b, adj)
    h = _project(y, g3_w, heads=heads, head_dim=out_c)
    h3 = _gat_mean(h, g3_as, g3_ad, g3_b, adj)

    return _fused_tail(adjt, h3, x, time_mean, p)
